# baseline (speedup 1.0000x reference)
import functools

import jax
import jax.numpy as jnp
from jax import lax
from jax.experimental import pallas as pl
from jax.experimental.pallas import tpu as pltpu
from jax.experimental.pallas import tpu_sc as plsc

LANES = 16


def _ds8(start, size):
    return pl.ds(pl.multiple_of(start, 8), size)


@functools.lru_cache(maxsize=None)
def _build(total, batch, n_fields, d, n_workers, n_cores):
    rows_per_worker = total // n_workers
    chunk_elems = 4
    chunk_rows = chunk_elems * n_fields
    n_chunks = rows_per_worker // chunk_rows
    pat = 208
    assert rows_per_worker % pat == 0

    mesh = plsc.VectorSubcoreMesh(core_axis_name="c", subcore_axis_name="s")

    @functools.partial(
        pl.kernel,
        mesh=mesh,
        out_type=jax.ShapeDtypeStruct((total, d), jnp.float32),
        scratch_types=[
            pltpu.VMEM((rows_per_worker,), jnp.int32),
            pltpu.VMEM((pat,), jnp.int32),
            pltpu.VMEM((n_fields * d,), jnp.float32),
            pltpu.VMEM((chunk_rows, d), jnp.float32),
            pltpu.SemaphoreType.DMA,
        ],
        compiler_params=pltpu.CompilerParams(use_tc_tiling_on_sc=False),
    )
    def emb_kernel(x_hbm, off_hbm, table_hbm, bias_hbm, out_hbm,
                   idx_v, off_v, bias_v, rows_v, sem):
        cid = lax.axis_index("c")
        sid = lax.axis_index("s")
        wid = sid * n_cores + cid
        wbase = wid * rows_per_worker

        pltpu.sync_copy(x_hbm.at[_ds8(wbase, rows_per_worker)], idx_v)
        pltpu.sync_copy(off_hbm, off_v)
        pltpu.sync_copy(bias_hbm, bias_v)

        def add_off(i, carry):
            base = i * pat
            for j in range(pat // LANES):
                sl = _ds8(base + j * LANES, LANES)
                idx_v[sl] = idx_v[sl] + off_v[pl.ds(j * LANES, LANES)]
            return carry
        lax.fori_loop(0, rows_per_worker // pat, add_off, 0)

        def chunk_body(c, carry):
            cbase = c * chunk_rows
            pltpu.async_copy(
                table_hbm.at[idx_v.at[_ds8(cbase, chunk_rows)]],
                rows_v, sem).wait()
            for f in range(n_fields):
                bv = [bias_v[pl.ds(f * d + j * LANES, LANES)]
                      for j in range(d // LANES)]
                for e in range(chunk_elems):
                    r = e * n_fields + f
                    for j in range(d // LANES):
                        sl = pl.ds(j * LANES, LANES)
                        rows_v[r, sl] = rows_v[r, sl] + bv[j]
            pltpu.sync_copy(rows_v, out_hbm.at[_ds8(wbase + cbase, chunk_rows)])
            return carry
        lax.fori_loop(0, n_chunks, chunk_body, 0)

    return emb_kernel


def kernel(x, table, bias, offsets):
    batch, n_fields = x.shape
    _, d = table.shape
    total = batch * n_fields

    info = plsc.get_sparse_core_info()
    n_workers = info.num_cores * info.num_subcores

    x_flat = x.reshape(-1).astype(jnp.int32)
    off_rep = jnp.tile(offsets.astype(jnp.int32), 208 // n_fields)
    bias_flat = bias.reshape(-1)

    out = _build(total, batch, n_fields, d, n_workers, info.num_cores)(
        x_flat, off_rep, table, bias_flat)
    return out.reshape(batch, n_fields, d)
